# Initial kernel scaffold; baseline (speedup 1.0000x reference)
import jax
import jax.numpy as jnp
from jax.experimental import pallas as pl


def kernel(edge_index, edge_type, emb_ent, emb_rel, Wn0, Wn1, Wl0, Wl1, Wel0, Wel1, time_gate_weight, time_gate_bias):
    raise NotImplementedError("write your pallas kernel here")


# speedup vs baseline: 2.3661x; 2.3661x over previous
import functools

import jax
import jax.numpy as jnp
from jax import lax
from jax.experimental import pallas as pl
from jax.experimental.pallas import tpu as pltpu
from jax.experimental.pallas import tpu_sc as plsc

N_ENT = 10000
H = 128
E = 320000
T = 3
NREL = 500
SLOPE = (1.0 / 8.0 + 1.0 / 3.0) / 2.0

NC, NS = 2, 16
NTILE = NC * NS
BATCH = 128
K = 80
E_PAD = NTILE * K * BATCH
N_PAD = 10240
ROWS_PER_SUB = N_PAD // NS
DUMMY_DST = N_ENT

def _make_segsum(gather):
    mesh = plsc.VectorSubcoreMesh(core_axis_name="c", subcore_axis_name="s",
                                  num_cores=NC, num_subcores=NS)

    @functools.partial(
        pl.kernel,
        out_type=jax.ShapeDtypeStruct((NC, N_PAD, H), jnp.float32),
        mesh=mesh,
        scratch_types=[
            pltpu.VMEM((K, BATCH), jnp.int32),
            pltpu.VMEM((K, BATCH), jnp.int32),
            pltpu.VMEM((BATCH, H), jnp.float32),
            pltpu.VMEM_SHARED((N_PAD, H), jnp.float32),
        ],
    )
    def segsum(table_hbm, src_hbm, dst_hbm, zeros_hbm, out_hbm,
               src_v, dst_v, rows_v, acc_sh):
        c = lax.axis_index("c")
        s = lax.axis_index("s")
        tile = s * NC + c
        sl = pl.ds(s * ROWS_PER_SUB, ROWS_PER_SUB)
        pltpu.sync_copy(zeros_hbm.at[sl], acc_sh.at[sl])
        base = tile * K
        if gather:
            pltpu.sync_copy(src_hbm.at[pl.ds(base, K)], src_v)
        else:
            pltpu.sync_copy(table_hbm, rows_v)
        pltpu.sync_copy(dst_hbm.at[pl.ds(base, K)], dst_v)
        plsc.subcore_barrier()

        @pl.loop(0, K)
        def _(j):
            if gather:
                pltpu.sync_copy(table_hbm.at[src_v.at[j]], rows_v)
            pltpu.sync_copy(rows_v, acc_sh.at[dst_v.at[j]], add=True)

        plsc.subcore_barrier()
        pltpu.sync_copy(acc_sh.at[sl], out_hbm.at[c].at[sl])

    return segsum


_segsum_cache = {}


def _segsum(gather, *args):
    if gather not in _segsum_cache:
        _segsum_cache[gather] = _make_segsum(gather)
    return _segsum_cache[gather](*args)


_BR = 1280
_G = N_PAD // _BR


def _l2norm_body(x_ref, o_ref):
    x = x_ref[...]
    n = jnp.sqrt(jnp.sum(x * x, axis=-1, keepdims=True))
    o_ref[...] = x / jnp.maximum(n, 1e-12)


_l2norm = pl.pallas_call(
    _l2norm_body,
    out_shape=jax.ShapeDtypeStruct((N_PAD, H), jnp.float32),
    grid=(_G,),
    in_specs=[pl.BlockSpec((_BR, H), lambda i: (i, 0))],
    out_specs=pl.BlockSpec((_BR, H), lambda i: (i, 0)),
)


def _prep_body(rp_ref, dg_ref, rn_ref, nb_ref):
    r = rp_ref[0] + rp_ref[1]
    deg = dg_ref[0][:, :1] + dg_ref[1][:, :1]
    norm = jnp.where(deg > 0, 1.0 / jnp.maximum(deg, 1.0), 0.0)
    rn_ref[...] = r * norm
    nb_ref[...] = jnp.broadcast_to(norm, (_BR, H))


_prep = pl.pallas_call(
    _prep_body,
    out_shape=(jax.ShapeDtypeStruct((N_PAD, H), jnp.float32),
               jax.ShapeDtypeStruct((N_PAD, H), jnp.float32)),
    grid=(_G,),
    in_specs=[pl.BlockSpec((NC, _BR, H), lambda i: (0, i, 0)),
              pl.BlockSpec((NC, _BR, H), lambda i: (0, i, 0))],
    out_specs=(pl.BlockSpec((_BR, H), lambda i: (i, 0)),
               pl.BlockSpec((_BR, H), lambda i: (i, 0))),
)


def _dot(a, b):
    return lax.dot_general(a, b, (((1,), (0,)), ((), ())),
                           precision=lax.Precision.HIGHEST,
                           preferred_element_type=jnp.float32)


def _layer_core(sp_ref, rn_ref, nb_ref, h_ref, wn_ref, wl_ref, wel_ref):
    S = sp_ref[0] + sp_ref[1]
    nb = nb_ref[...]
    X = S * nb + rn_ref[...]
    h = h_ref[...]
    Y = _dot(X, wn_ref[...]) + jnp.where(
        nb > 0, _dot(h, wl_ref[...]), _dot(h, wel_ref[...]))
    return jnp.where(Y >= 0, Y, SLOPE * Y)


def _layer_body(sp_ref, rn_ref, nb_ref, h_ref, wn_ref, wl_ref, wel_ref, o_ref):
    o_ref[...] = _layer_core(sp_ref, rn_ref, nb_ref, h_ref,
                             wn_ref, wl_ref, wel_ref)


def _layer2_body(sp_ref, rn_ref, nb_ref, h_ref, wn_ref, wl_ref, wel_ref,
                 ht_ref, tg_ref, tb_ref, o_ref):
    cur = _layer_core(sp_ref, rn_ref, nb_ref, h_ref, wn_ref, wl_ref, wel_ref)
    n = jnp.sqrt(jnp.sum(cur * cur, axis=-1, keepdims=True))
    curn = cur / jnp.maximum(n, 1e-12)
    ht = ht_ref[...]
    tw = jax.nn.sigmoid(_dot(ht, tg_ref[...]) + tb_ref[...])
    o_ref[...] = tw * curn + (1.0 - tw) * ht


_row_spec = pl.BlockSpec((_BR, H), lambda i: (i, 0))
_w_spec = pl.BlockSpec((H, H), lambda i: (0, 0))
_sp_spec = pl.BlockSpec((NC, _BR, H), lambda i: (0, i, 0))

_layer = pl.pallas_call(
    _layer_body,
    out_shape=jax.ShapeDtypeStruct((N_PAD, H), jnp.float32),
    grid=(_G,),
    in_specs=[_sp_spec, _row_spec, _row_spec, _row_spec,
              _w_spec, _w_spec, _w_spec],
    out_specs=_row_spec,
)

_layer2 = pl.pallas_call(
    _layer2_body,
    out_shape=jax.ShapeDtypeStruct((N_PAD, H), jnp.float32),
    grid=(_G,),
    in_specs=[_sp_spec, _row_spec, _row_spec, _row_spec,
              _w_spec, _w_spec, _w_spec,
              _row_spec, _w_spec, pl.BlockSpec((1, H), lambda i: (0, 0))],
    out_specs=_row_spec,
)


def kernel(edge_index, edge_type, emb_ent, emb_rel, Wn0, Wn1, Wl0, Wl1,
           Wel0, Wel1, time_gate_weight, time_gate_bias):
    pad_e = E_PAD - E
    src = jnp.pad(edge_index[:, 0, :].astype(jnp.int32),
                  ((0, 0), (0, pad_e))).reshape(T, -1, BATCH)
    dst = jnp.pad(edge_index[:, 1, :].astype(jnp.int32),
                  ((0, 0), (0, pad_e)),
                  constant_values=DUMMY_DST).reshape(T, -1, BATCH)
    et = jnp.pad(edge_type.astype(jnp.int32),
                 ((0, 0), (0, pad_e))).reshape(T, -1, BATCH)

    z128 = jnp.zeros((N_PAD, H), jnp.float32)
    ones_rows = jnp.ones((BATCH, H), jnp.float32)
    emb_pad = jnp.pad(emb_ent, ((0, N_PAD - N_ENT), (0, 0)))
    tb = time_gate_bias.reshape(1, H)

    h = _l2norm(emb_pad)
    hist = [h]
    Wns = (Wn0, Wn1)
    Wls = (Wl0, Wl1)
    Wels = (Wel0, Wel1)
    for t in range(T):
        rp = _segsum(True, emb_rel, et[t], dst[t], z128)
        dg = _segsum(False, ones_rows, et[t], dst[t], z128)
        rn, nb = _prep(rp, dg)
        ht = h
        cur = ht
        sp = _segsum(True, cur, src[t], dst[t], z128)
        cur = _layer(sp, rn, nb, cur, Wns[0], Wls[0], Wels[0])
        sp = _segsum(True, cur, src[t], dst[t], z128)
        h = _layer2(sp, rn, nb, cur, Wns[1], Wls[1], Wels[1],
                    ht, time_gate_weight, tb)
        hist.append(h)
    return jnp.stack(hist, axis=0)[:, :N_ENT, :]
